# Initial kernel scaffold; baseline (speedup 1.0000x reference)
import jax
import jax.numpy as jnp
from jax.experimental import pallas as pl


def kernel(start, mid, end, durations, max_frames):
    raise NotImplementedError("write your pallas kernel here")


# speedup vs baseline: 8.7545x; 8.7545x over previous
import functools

import jax
import jax.numpy as jnp
from jax import lax
from jax.experimental import pallas as pl
from jax.experimental.pallas import tpu as pltpu
from jax.experimental.pallas import tpu_sc as plsc

B, N, F = 8, 512, 512
T = 7 * N
ZERO_ROW = B * 3 * N
TBL_ROWS = ZERO_ROW + 8
NQ = 4
TQ = T // NQ
CHUNK = 112
NCHUNK = TQ // CHUNK
VPB = N // 16
VPT = T // 16


def _body(tbl_hbm, dur_hbm, out_hbm, mask_hbm, dur_v, idx_full, mask_v, rows, sem):
    cid = lax.axis_index("c")
    sid = lax.axis_index("s")
    b = cid * 4 + sid // NQ
    q = sid % NQ

    pltpu.sync_copy(dur_hbm.at[b], dur_v)

    def init_body(i, _):
        idx_full[pl.ds(i * 16, 16)] = jnp.full((16,), ZERO_ROW, jnp.int32)
        return _
    lax.fori_loop(0, VPT, init_body, 0)

    base_b = b * (3 * N)
    lane = lax.iota(jnp.int32, 16)

    def seg_body(i, carry):
        d = dur_v[pl.ds(i * 16, 16)]
        o = plsc.cumsum(d) - d + carry
        n = base_b + i * 16 + lane
        vmid = n + N
        for j in range(7):
            if j == 0:
                val = jnp.where(d >= 2, n, vmid)
            else:
                val = jnp.where(d == j + 1, n + 2 * N, vmid)
            plsc.store_scatter(idx_full, [o + j], val, mask=d > j)
        return carry + jnp.sum(d)
    total = lax.fori_loop(0, VPB, seg_body, jnp.int32(0))

    def mask_body(i, _):
        t16 = i * 16 + lane
        mask_v[pl.ds(i * 16, 16)] = jnp.where(t16 < total, 1, 0)
        return _
    lax.fori_loop(0, VPT, mask_body, 0)

    @pl.when(q == 0)
    def _():
        pltpu.sync_copy(mask_v, mask_hbm.at[b])

    def gather_start(c, buf):
        idxs = idx_full.at[pl.ds(q * TQ + c * CHUNK, CHUNK)]
        return pltpu.async_copy(tbl_hbm.at[idxs], rows.at[buf], sem)

    descs = [gather_start(0, 0)]
    row_base = b * T + q * TQ
    for c in range(NCHUNK):
        descs[c].wait()
        if c + 1 < NCHUNK:
            descs.append(gather_start(c + 1, (c + 1) % 2))
        pltpu.sync_copy(rows.at[c % 2], out_hbm.at[pl.ds(row_base + c * CHUNK, CHUNK)])


@jax.jit
def _hmm_interp(table, durations):
    mesh = plsc.VectorSubcoreMesh(
        core_axis_name="c", subcore_axis_name="s", num_cores=2, num_subcores=16)
    run = pl.kernel(
        _body,
        out_type=(
            jax.ShapeDtypeStruct((B * T, F), jnp.float32),
            jax.ShapeDtypeStruct((B, T), jnp.int32),
        ),
        mesh=mesh,
        scratch_types=[
            pltpu.VMEM((N,), jnp.int32),
            pltpu.VMEM((T,), jnp.int32),
            pltpu.VMEM((T,), jnp.int32),
            pltpu.VMEM((2, CHUNK, F), jnp.float32),
            pltpu.SemaphoreType.DMA,
        ],
        compiler_params=pltpu.CompilerParams(needs_layout_passes=False),
    )
    return run(table, durations)


def kernel(start, mid, end, durations, max_frames):
    table = jnp.concatenate([start, mid, end], axis=1).reshape(B * 3 * N, F)
    table = jnp.pad(table, ((0, TBL_ROWS - ZERO_ROW), (0, 0)))
    out_flat, mask_i32 = _hmm_interp(table, durations)
    return out_flat.reshape(B, T, F), mask_i32.astype(jnp.bool_)
